# baseline (speedup 1.0000x reference)
import dataclasses

import jax
import jax.numpy as jnp
from jax import lax
from jax.experimental import pallas as pl
from jax.experimental.pallas import tpu as pltpu
from jax.experimental.pallas import tpu_sc as plsc

H = 128
W = 128
C = 32
HO = 256
WO = 256
F = 3
NW = 32
ROWS_PER_W = HO // NW
QTAP = 12
XROWS = 6


def _pat(shift, lo=0, hi=15):
    i = jnp.arange(16) // 2 + shift
    return jnp.clip(i, lo, hi).astype(jnp.int32)


def _sc_kernel(xm_hbm, xa_hbm, w_hbm, out_hbm, xm, xa, wb0, wb1, wb2, obuf,
               sem_x, sem_w0, sem_w1, sem_w2, sem_o):
    nc = 2
    wid = lax.axis_index("s") * nc + lax.axis_index("c")
    wbufs = (wb0, wb1, wb2)
    wsems = (sem_w0, sem_w1, sem_w2)

    pltpu.async_copy(xm_hbm.at[pl.ds(wid * 4, XROWS)], xm, sem_x).wait()
    pltpu.async_copy(xa_hbm.at[pl.ds(wid * 4, XROWS)], xa, sem_x).wait()

    def w_slice(r_out, tap):
        return w_hbm.at[r_out, pl.ds(QTAP * tap, QTAP)]

    def out_copies(rr2):
        r0 = wid * ROWS_PER_W + 2 * rr2
        i8 = lax.div(r0, jnp.int32(8))
        is0 = lax.rem(r0, jnp.int32(8))
        cps = []
        for f in range(F):
            for jt in range(2):
                cps.append(pltpu.make_async_copy(
                    obuf.at[pl.ds((f * 2 + jt) * 256, 256)],
                    out_hbm.at[f, i8, jt, pl.ds(is0 * 128, 256)],
                    sem_o))
        return cps

    def group_compute(tap, rr, gp, buf, peel):
        dr, dc = tap // 3, tap % 3
        xrow = lax.div(rr, jnp.int32(2))
        rhalf = lax.rem(rr, jnp.int32(2))
        if peel:
            jt, jl = 1, 96
            colbase = 112
        else:
            jt = lax.div(gp, jnp.int32(4))
            jl = gp * 32 - jt * 128
            colbase = gp * 16

        def aoff(e2, f):
            return ((f * 2 + jt) * 2 + rhalf) * 128 + jl + e2 * 16

        acc = []
        for e2 in range(2):
            for f in range(F):
                if tap == 0:
                    acc.append(jnp.zeros((16,), jnp.float32))
                else:
                    acc.append(obuf[pl.ds(aoff(e2, f), 16)])

        patA = _pat(dc)
        patB0 = _pat(8)
        patBlo = _pat(8 + dc)
        patBhi = _pat(8 + dc - 16, lo=0)
        selB = (jnp.arange(16) // 2 + 8 + dc) > 15

        def qbody(Q, accs):
            accs = list(accs)
            for cc in range(8):
                c = 8 * Q + cc
                xv = xm[xrow + dr, c, pl.ds(colbase, 16)]
                xpA = xv.at[patA].get(mode="promise_in_bounds")
                if peel:
                    c8 = lax.div(c, jnp.int32(8))
                    co = (c - c8 * 8) * 16
                    xv2 = xa[xrow + dr, c8, pl.ds(co, 16)]
                    xpB = xv2.at[patA].get(mode="promise_in_bounds")
                elif dc == 0:
                    xpB = xv.at[patB0].get(mode="promise_in_bounds")
                else:
                    xv2 = xm[xrow + dr, c, pl.ds(colbase + 16, 16)]
                    blo = xv.at[patBlo].get(mode="promise_in_bounds")
                    bhi = xv2.at[patBhi].get(mode="promise_in_bounds")
                    xpB = jnp.where(selB, bhi, blo)
                for f in range(F):
                    qq, s = divmod(3 * cc + f, 8)
                    for e2, xp in ((0, xpA), (1, xpB)):
                        wv = buf[3 * Q + qq, jt, s, pl.ds(jl + e2 * 16, 16)]
                        accs[e2 * 3 + f] = accs[e2 * 3 + f] + xp * wv
            return tuple(accs)

        acc = list(lax.fori_loop(0, 4, qbody, tuple(acc)))

        for e2 in range(2):
            for f in range(F):
                obuf[pl.ds(aoff(e2, f), 16)] = acc[e2 * 3 + f]

    pltpu.make_async_copy(w_slice(wid * ROWS_PER_W, 0), wb0, sem_w0).start()
    pltpu.make_async_copy(w_slice(wid * ROWS_PER_W, 1), wb1, sem_w1).start()

    @pl.loop(0, ROWS_PER_W)
    def _(rr):
        r_out = wid * ROWS_PER_W + rr

        @pl.when((lax.rem(rr, jnp.int32(2)) == 0) & (rr > 0))
        def _():
            for cp in out_copies(lax.div(rr, jnp.int32(2)) - 1):
                cp.wait()

        for tap in range(9):
            buf, sem = wbufs[tap % 3], wsems[tap % 3]
            pltpu.make_async_copy(w_slice(r_out, tap), buf, sem).wait()
            nb, ns = wbufs[(tap + 2) % 3], wsems[(tap + 2) % 3]
            if tap < 7:
                pltpu.make_async_copy(w_slice(r_out, tap + 2), nb, ns).start()
            else:
                @pl.when(rr < ROWS_PER_W - 1)
                def _(_tap=tap):
                    pltpu.make_async_copy(
                        w_slice(r_out + 1, _tap - 7), nb, ns).start()

            @pl.loop(0, 7)
            def _(gp):
                group_compute(tap, rr, gp, buf, peel=False)

            group_compute(tap, rr, jnp.int32(7), buf, peel=True)

        @pl.when(lax.rem(rr, jnp.int32(2)) == 1)
        def _():
            for cp in out_copies(lax.div(rr, jnp.int32(2))):
                cp.start()

    for cp in out_copies(jnp.int32(ROWS_PER_W // 2 - 1)):
        cp.wait()


@jax.jit
def kernel(x, meta_w):
    w5 = (meta_w[0].transpose(0, 2, 1).reshape(HO, 108, 8, 2, 128)
          .transpose(0, 1, 3, 2, 4))
    xt = jnp.transpose(x[0], (0, 2, 1))
    xp = jnp.pad(xt, ((1, 1), (0, 0), (1, 7)))
    xm_in = xp[:, :, :128]
    xa_in = jnp.pad(xp[:, :, 120:136].reshape(130, 4, 128),
                    ((0, 0), (0, 4), (0, 0)))

    mesh = plsc.VectorSubcoreMesh(core_axis_name="c", subcore_axis_name="s")
    cp = pltpu.CompilerParams()
    if "needs_layout_passes" in pltpu.CompilerParams.__dataclass_fields__:
        cp = dataclasses.replace(cp, needs_layout_passes=False)
    out = pl.kernel(
        _sc_kernel,
        mesh=mesh,
        compiler_params=cp,
        out_type=jax.ShapeDtypeStruct((F, HO // 8, 2, 8 * 128), jnp.float32),
        scratch_types=[
            pltpu.VMEM((XROWS, C, 128), jnp.float32),
            pltpu.VMEM((XROWS, 8, 128), jnp.float32),
            pltpu.VMEM((QTAP, 2, 8, 128), jnp.float32),
            pltpu.VMEM((QTAP, 2, 8, 128), jnp.float32),
            pltpu.VMEM((QTAP, 2, 8, 128), jnp.float32),
            pltpu.VMEM((F * 2 * 2 * 128,), jnp.float32),
            pltpu.SemaphoreType.DMA,
            pltpu.SemaphoreType.DMA,
            pltpu.SemaphoreType.DMA,
            pltpu.SemaphoreType.DMA,
            pltpu.SemaphoreType.DMA,
        ],
    )(xm_in, xa_in, w5)
    o = out.reshape(F, HO // 8, 2, 8, 128).transpose(0, 1, 3, 2, 4)
    o = o.reshape(F, HO, WO).transpose(1, 2, 0)
    return o[None]


# speedup vs baseline: 1.0088x; 1.0088x over previous
import dataclasses

import jax
import jax.numpy as jnp
from jax import lax
from jax.experimental import pallas as pl
from jax.experimental.pallas import tpu as pltpu
from jax.experimental.pallas import tpu_sc as plsc

H = 128
W = 128
C = 32
HO = 256
WO = 256
F = 3
NW = 32
ROWS_PER_W = HO // NW
QTAP = 12
XROWS = 6


def _pat(shift, lo=0, hi=15):
    i = jnp.arange(16) // 2 + shift
    return jnp.clip(i, lo, hi).astype(jnp.int32)


def _sc_kernel(xm_hbm, xa_hbm, w_hbm, out_hbm, xm, xa, wb0, wb1, wb2, obuf,
               sem_x, sem_w0, sem_w1, sem_w2, sem_o):
    nc = 2
    wid = lax.axis_index("s") * nc + lax.axis_index("c")
    wbufs = (wb0, wb1, wb2)
    wsems = (sem_w0, sem_w1, sem_w2)

    cx1 = pltpu.make_async_copy(xm_hbm.at[pl.ds(wid * 4, XROWS)], xm, sem_x)
    cx2 = pltpu.make_async_copy(xa_hbm.at[pl.ds(wid * 4, XROWS)], xa, sem_x)
    cx1.start()
    cx2.start()
    cx1.wait()
    cx2.wait()

    def w_slice(r_out, tap):
        return w_hbm.at[r_out, pl.ds(QTAP * tap, QTAP)]

    def out_copies(rr2):
        r0 = wid * ROWS_PER_W + 2 * rr2
        i8 = lax.div(r0, jnp.int32(8))
        is0 = lax.rem(r0, jnp.int32(8))
        cps = []
        for f in range(F):
            for jt in range(2):
                cps.append(pltpu.make_async_copy(
                    obuf.at[pl.ds((f * 2 + jt) * 256, 256)],
                    out_hbm.at[f, i8, jt, pl.ds(is0 * 128, 256)],
                    sem_o))
        return cps

    def group_compute(tap, rr, gp, buf, peel):
        dr, dc = tap // 3, tap % 3
        xrow = lax.div(rr, jnp.int32(2))
        rhalf = lax.rem(rr, jnp.int32(2))
        if peel:
            jt, jl = 1, 96
            colbase = 112
        else:
            jt = lax.div(gp, jnp.int32(4))
            jl = gp * 32 - jt * 128
            colbase = gp * 16

        def aoff(e2, f):
            return ((f * 2 + jt) * 2 + rhalf) * 128 + jl + e2 * 16

        acc = []
        for e2 in range(2):
            for f in range(F):
                if tap == 0:
                    acc.append(jnp.zeros((16,), jnp.float32))
                else:
                    acc.append(obuf[pl.ds(aoff(e2, f), 16)])

        patA = _pat(dc)
        patB0 = _pat(8)
        patBlo = _pat(8 + dc)
        patBhi = _pat(8 + dc - 16, lo=0)
        selB = (jnp.arange(16) // 2 + 8 + dc) > 15

        def qbody(Q, accs):
            accs = list(accs)
            for cc in range(8):
                c = 8 * Q + cc
                xv = xm[xrow + dr, c, pl.ds(colbase, 16)]
                xpA = xv.at[patA].get(mode="promise_in_bounds")
                if peel:
                    c8 = lax.div(c, jnp.int32(8))
                    co = (c - c8 * 8) * 16
                    xv2 = xa[xrow + dr, c8, pl.ds(co, 16)]
                    xpB = xv2.at[patA].get(mode="promise_in_bounds")
                elif dc == 0:
                    xpB = xv.at[patB0].get(mode="promise_in_bounds")
                else:
                    xv2 = xm[xrow + dr, c, pl.ds(colbase + 16, 16)]
                    blo = xv.at[patBlo].get(mode="promise_in_bounds")
                    bhi = xv2.at[patBhi].get(mode="promise_in_bounds")
                    xpB = jnp.where(selB, bhi, blo)
                for f in range(F):
                    qq, s = divmod(3 * cc + f, 8)
                    for e2, xp in ((0, xpA), (1, xpB)):
                        wv = buf[3 * Q + qq, jt, s, pl.ds(jl + e2 * 16, 16)]
                        accs[e2 * 3 + f] = accs[e2 * 3 + f] + xp * wv
            return tuple(accs)

        acc = list(lax.fori_loop(0, 4, qbody, tuple(acc)))

        for e2 in range(2):
            for f in range(F):
                obuf[pl.ds(aoff(e2, f), 16)] = acc[e2 * 3 + f]

    pltpu.make_async_copy(w_slice(wid * ROWS_PER_W, 0), wb0, sem_w0).start()
    pltpu.make_async_copy(w_slice(wid * ROWS_PER_W, 1), wb1, sem_w1).start()

    @pl.loop(0, ROWS_PER_W)
    def _(rr):
        r_out = wid * ROWS_PER_W + rr

        @pl.when((lax.rem(rr, jnp.int32(2)) == 0) & (rr > 0))
        def _():
            for cp in out_copies(lax.div(rr, jnp.int32(2)) - 1):
                cp.wait()

        for tap in range(9):
            buf, sem = wbufs[tap % 3], wsems[tap % 3]
            pltpu.make_async_copy(w_slice(r_out, tap), buf, sem).wait()
            nb, ns = wbufs[(tap + 2) % 3], wsems[(tap + 2) % 3]
            if tap < 7:
                pltpu.make_async_copy(w_slice(r_out, tap + 2), nb, ns).start()
            else:
                @pl.when(rr < ROWS_PER_W - 1)
                def _(_tap=tap):
                    pltpu.make_async_copy(
                        w_slice(r_out + 1, _tap - 7), nb, ns).start()

            @pl.loop(0, 7)
            def _(gp):
                group_compute(tap, rr, gp, buf, peel=False)

            group_compute(tap, rr, jnp.int32(7), buf, peel=True)

        @pl.when(lax.rem(rr, jnp.int32(2)) == 1)
        def _():
            for cp in out_copies(lax.div(rr, jnp.int32(2))):
                cp.start()

    for cp in out_copies(jnp.int32(ROWS_PER_W // 2 - 1)):
        cp.wait()


@jax.jit
def kernel(x, meta_w):
    w5 = (meta_w[0].transpose(0, 2, 1).reshape(HO, 108, 8, 2, 128)
          .transpose(0, 1, 3, 2, 4))
    xt = jnp.transpose(x[0], (0, 2, 1))
    xp = jnp.pad(xt, ((1, 1), (0, 0), (1, 7)))
    xm_in = xp[:, :, :128]
    xa_in = jnp.pad(xp[:, :, 120:136].reshape(130, 4, 128),
                    ((0, 0), (0, 4), (0, 0)))

    mesh = plsc.VectorSubcoreMesh(core_axis_name="c", subcore_axis_name="s")
    cp = pltpu.CompilerParams()
    if "needs_layout_passes" in pltpu.CompilerParams.__dataclass_fields__:
        cp = dataclasses.replace(cp, needs_layout_passes=False)
    out = pl.kernel(
        _sc_kernel,
        mesh=mesh,
        compiler_params=cp,
        out_type=jax.ShapeDtypeStruct((F, HO // 8, 2, 8 * 128), jnp.float32),
        scratch_types=[
            pltpu.VMEM((XROWS, C, 128), jnp.float32),
            pltpu.VMEM((XROWS, 8, 128), jnp.float32),
            pltpu.VMEM((QTAP, 2, 8, 128), jnp.float32),
            pltpu.VMEM((QTAP, 2, 8, 128), jnp.float32),
            pltpu.VMEM((QTAP, 2, 8, 128), jnp.float32),
            pltpu.VMEM((F * 2 * 2 * 128,), jnp.float32),
            pltpu.SemaphoreType.DMA,
            pltpu.SemaphoreType.DMA,
            pltpu.SemaphoreType.DMA,
            pltpu.SemaphoreType.DMA,
            pltpu.SemaphoreType.DMA,
        ],
    )(xm_in, xa_in, w5)
    o = out.reshape(F, HO // 8, 2, 8, 128).transpose(0, 1, 3, 2, 4)
    o = o.reshape(F, HO, WO).transpose(1, 2, 0)
    return o[None]
